# baseline (speedup 1.0000x reference)
import functools
import jax
import jax.numpy as jnp
from jax import lax
from jax.experimental import pallas as pl
from jax.experimental.pallas import tpu as pltpu
from jax.experimental.pallas import tpu_sc as plsc

N = 100000
E = 3200000
CHUNK = 128
BLK = 640
NCHUNK = BLK // CHUNK
E_PAD = 3276800
NCH_TOT = E_PAD // CHUNK
EP_TILE = E_PAD // 16
NB = EP_TILE // BLK
NB4 = NB // 4
NS = 102400
RPT = NS // 16
_ZSP = tuple((k * BLK, BLK) for k in range(10))
_ZDG = tuple((k * 1024, 1024) for k in range(6)) + ((6144, 256),)

DBLK = 1024
DNCH = DBLK // CHUNK
EP_DEG = E_PAD // 32
NB_DEG = EP_DEG // DBLK

TBLK = 1024
TNCH = TBLK // CHUNK

ROWS = 800
NT = N // ROWS
NST = NS // ROWS


def _deg_body(sd_hbm, out_hbm, ones_v, zero_v, idx_v, acc_sh):
    c = lax.axis_index("c")
    tid = lax.axis_index("s")
    row0 = tid * RPT

    def init(k, carry):
        ones_v[pl.ds(k * 16, 16)] = jnp.ones((16,), jnp.float32)
        zero_v[pl.ds(k * 16, 16)] = jnp.zeros((16,), jnp.float32)
        return carry
    lax.fori_loop(0, DBLK // 16, init, 0)

    for k0, sz in _ZDG:
        pltpu.sync_copy(zero_v.at[pl.ds(0, sz)], acc_sh.at[pl.ds(row0 + k0, sz)])
    plsc.subcore_barrier()

    def block(i, carry):
        k0 = (c * 16 + tid) * (EP_DEG // CHUNK) + i * DNCH
        pltpu.sync_copy(sd_hbm.at[pl.ds(2 * k0, 2 * DNCH)], idx_v)
        for j in range(DNCH):
            pltpu.sync_copy(ones_v.at[pl.ds(j * CHUNK, CHUNK)],
                            acc_sh.at[idx_v.at[2 * j + 1]], add=True)
        return carry
    lax.fori_loop(0, NB_DEG, block, 0)
    plsc.subcore_barrier()

    for k0, sz in _ZDG:
        pltpu.sync_copy(acc_sh.at[pl.ds(row0 + k0, sz)],
                        out_hbm.at[pl.ds(c * NS + row0 + k0, sz)])


def _sc_degree(sd1):
    kern = pl.kernel(
        _deg_body,
        out_type=jax.ShapeDtypeStruct((2 * NS,), jnp.float32),
        mesh=plsc.VectorSubcoreMesh(core_axis_name="c", subcore_axis_name="s"),
        scratch_types=[
            pltpu.VMEM((DBLK,), jnp.float32),
            pltpu.VMEM((DBLK,), jnp.float32),
            pltpu.VMEM((2 * DNCH, CHUNK), jnp.int32),
            pltpu.VMEM_SHARED((NS,), jnp.float32),
        ],
        compiler_params=pltpu.CompilerParams(use_tc_tiling_on_sc=False),
    )
    return kern(sd1)


def _spmm_body(spc, g_hbm, sd_hbm, out_hbm,
               idx_v, rows_v, acc_sh, stg0, stg1, gs0, gs1, ss0, ss1):
    c = lax.axis_index("c")
    tid = lax.axis_index("s")
    row0 = tid * RPT
    nsl = 2 * spc
    stg = (stg0, stg1)
    gs = (gs0, gs1)
    ss = (ss0, ss1)
    ch0 = tid * (EP_TILE // CHUNK)

    def fire_stage(bi, q, p):
        k0 = ch0 + bi * NCHUNK
        pltpu.async_copy(sd_hbm.at[pl.ds(2 * k0, 2 * NCHUNK)],
                         idx_v.at[q], stg[p])

    def wait_stage(q, p):
        pltpu.make_async_copy(sd_hbm.at[pl.ds(0, 2 * NCHUNK)],
                              idx_v.at[q], stg[p]).wait()

    def fire_gathers(gb, q, r, p):
        for j in range(NCHUNK):
            pltpu.async_copy(gb.at[idx_v.at[q, 2 * j]],
                             rows_v.at[r, pl.ds(j * CHUNK, CHUNK)], gs[p])

    def wait_gathers(r, p):
        pltpu.make_async_copy(g_hbm.at[pl.ds(0, BLK)],
                              rows_v.at[r], gs[p]).wait()

    def fire_scatters(q, r, p):
        for j in range(NCHUNK):
            pltpu.async_copy(rows_v.at[r, pl.ds(j * CHUNK, CHUNK)],
                             acc_sh.at[idx_v.at[q, 2 * j + 1]], ss[p],
                             add=True)

    def wait_scatters(r, p):
        pltpu.make_async_copy(g_hbm.at[pl.ds(0, BLK)],
                              rows_v.at[r], ss[p]).wait()

    for sl in range(spc):
        slice_id = c * spc + sl
        gb = g_hbm
        offv = jnp.full((16,), slice_id, jnp.int32)

        def add_off(q):
            for jrow in range(NCHUNK):
                def aok(k, cc):
                    idx_v[q, 2 * jrow, pl.ds(k * 16, 16)] = (
                        idx_v[q, 2 * jrow, pl.ds(k * 16, 16)] + offv)
                    return cc
                lax.fori_loop(0, CHUNK // 16, aok, 0)

        def zb(k, carry):
            rows_v[0, k, :] = jnp.zeros((16,), jnp.float32)
            return carry
        lax.fori_loop(0, BLK, zb, 0)
        for k0, sz in _ZSP:
            pltpu.sync_copy(rows_v.at[0, pl.ds(0, sz)],
                            acc_sh.at[pl.ds(row0 + k0, sz)])
        plsc.subcore_barrier()

        fire_stage(0, 0, 0)
        fire_stage(1, 1, 1)

        def body(ii, carry):
            for jj in range(4):
                bi = 4 * ii + jj
                q, r, p = jj, jj % 2, jj % 2
                qp, rp, pp = (jj - 1) % 4, (jj - 1) % 2, (jj - 1) % 2
                if jj >= 2:
                    wait_scatters(r, p)
                else:
                    @pl.when(ii > 0)
                    def _():
                        wait_scatters(r, p)
                wait_stage(q, p)
                add_off(q)
                if jj < 2:
                    fire_stage(bi + 2, (jj + 2) % 4, p)
                else:
                    @pl.when(ii < NB4 - 1)
                    def _():
                        fire_stage(bi + 2, (jj + 2) % 4, p)
                fire_gathers(gb, q, r, p)
                if jj >= 1:
                    wait_gathers(rp, pp)
                    fire_scatters(qp, rp, pp)
                else:
                    @pl.when(ii > 0)
                    def _():
                        wait_gathers(rp, pp)
                        fire_scatters(qp, rp, pp)
            return carry
        lax.fori_loop(0, NB4, body, 0)

        wait_gathers((NB - 1) % 2, (NB - 1) % 2)
        fire_scatters((NB - 1) % 4, (NB - 1) % 2, (NB - 1) % 2)
        wait_scatters((NB - 2) % 2, (NB - 2) % 2)
        wait_scatters((NB - 1) % 2, (NB - 1) % 2)
        plsc.subcore_barrier()

        out_base = slice_id * NS + row0
        for k0, sz in _ZSP:
            pltpu.sync_copy(acc_sh.at[pl.ds(row0 + k0, sz)],
                            out_hbm.at[pl.ds(out_base + k0, sz)])
        if sl + 1 < spc:
            plsc.subcore_barrier()


def _sc_spmm(g_blocked, sd, n_slices):
    spc = n_slices // 2
    kern = pl.kernel(
        functools.partial(_spmm_body, spc),
        out_type=jax.ShapeDtypeStruct((n_slices * NS, 16), jnp.float32),
        mesh=plsc.VectorSubcoreMesh(core_axis_name="c", subcore_axis_name="s"),
        scratch_types=[
            pltpu.VMEM((4, 2 * NCHUNK, CHUNK), jnp.int32),
            pltpu.VMEM((2, BLK, 16), jnp.float32),
            pltpu.VMEM_SHARED((NS, 16), jnp.float32),
            pltpu.SemaphoreType.DMA,
            pltpu.SemaphoreType.DMA,
            pltpu.SemaphoreType.DMA,
            pltpu.SemaphoreType.DMA,
            pltpu.SemaphoreType.DMA,
            pltpu.SemaphoreType.DMA,
        ],
        compiler_params=pltpu.CompilerParams(use_tc_tiling_on_sc=False),
    )
    return kern(g_blocked, sd)


def _tgt_body(z_hbm, t_hbm, out_hbm, idx_v, rows_v, sem):
    c = lax.axis_index("c")
    tid = lax.axis_index("s")
    base = (c * 16 + tid) * TBLK
    pltpu.sync_copy(t_hbm.at[pl.ds(base, TBLK)], idx_v)
    handles = []
    for j in range(TNCH):
        handles.append(pltpu.async_copy(
            z_hbm.at[idx_v.at[pl.ds(j * CHUNK, CHUNK)]],
            rows_v.at[pl.ds(j * CHUNK, CHUNK)], sem))
    for h in handles:
        h.wait()
    pltpu.sync_copy(rows_v, out_hbm.at[pl.ds(base, TBLK)])


def _sc_gather_targets(z, tflat):
    kern = pl.kernel(
        _tgt_body,
        out_type=jax.ShapeDtypeStruct((32768, 32), jnp.float32),
        mesh=plsc.VectorSubcoreMesh(core_axis_name="c", subcore_axis_name="s"),
        scratch_types=[
            pltpu.VMEM((TBLK,), jnp.int32),
            pltpu.VMEM((TBLK, 32), jnp.float32),
            pltpu.SemaphoreType.DMA,
        ],
        compiler_params=pltpu.CompilerParams(use_tc_tiling_on_sc=False),
    )
    return kern(z, tflat)


def _k0_body(s_ref, d_ref, sd1_ref, sd2_ref):
    s = s_ref[...]
    d = d_ref[...]
    sd1_ref[...] = jnp.stack([s * 4, d], axis=1).reshape(sd1_ref.shape)
    sd2_ref[...] = jnp.stack([s * 2, d], axis=1).reshape(sd2_ref.shape)


def _tc_idx_build(src2d, dst2d):
    rb = 800
    g = NCH_TOT // rb
    return pl.pallas_call(
        _k0_body,
        grid=(g,),
        in_specs=[
            pl.BlockSpec((rb, CHUNK), lambda t: (t, 0)),
            pl.BlockSpec((rb, CHUNK), lambda t: (t, 0)),
        ],
        out_specs=[
            pl.BlockSpec((2 * rb, CHUNK), lambda t: (t, 0)),
            pl.BlockSpec((2 * rb, CHUNK), lambda t: (t, 0)),
        ],
        out_shape=[
            jax.ShapeDtypeStruct((2 * NCH_TOT, CHUNK), jnp.int32),
            jax.ShapeDtypeStruct((2 * NCH_TOT, CHUNK), jnp.int32),
        ],
    )(src2d, dst2d)


def _k2_body(x_ref, w_ref, d0_ref, d1_ref, g_ref, dinv_ref):
    dinv = lax.rsqrt(d0_ref[...] + d1_ref[...] + 1.0)
    g_ref[...] = jnp.dot(x_ref[...], w_ref[...],
                         preferred_element_type=jnp.float32) * dinv
    dinv_ref[...] = dinv


def _tc_scale_l1(x, W1, degpart):
    return pl.pallas_call(
        _k2_body,
        grid=(NT,),
        in_specs=[
            pl.BlockSpec((ROWS, 18), lambda t: (t, 0)),
            pl.BlockSpec((18, 64), lambda t: (0, 0)),
            pl.BlockSpec((ROWS, 1), lambda t: (t, 0)),
            pl.BlockSpec((ROWS, 1), lambda t: (NST + t, 0)),
        ],
        out_specs=[
            pl.BlockSpec((ROWS, 64), lambda t: (t, 0)),
            pl.BlockSpec((ROWS, 1), lambda t: (t, 0)),
        ],
        out_shape=[
            jax.ShapeDtypeStruct((N, 64), jnp.float32),
            jax.ShapeDtypeStruct((N, 1), jnp.float32),
        ],
    )(x, W1, degpart, degpart)


def _k4_body(a0, a1, a2, a3, g_ref, dinv_ref, b1_ref, w_ref, out_ref):
    dinv = dinv_ref[...]
    acc = jnp.concatenate([a[...] for a in (a0, a1, a2, a3)], axis=1)
    h = jax.nn.relu(dinv * (acc + g_ref[...]) + b1_ref[...])
    out_ref[...] = jnp.dot(h, w_ref[...], preferred_element_type=jnp.float32) * dinv


def _tc_layer2_tables(acc1, g1flat, dinv, b1, W2):
    in_acc = [pl.BlockSpec((ROWS, 16), functools.partial(
        lambda s, t: (s * NST + t, 0), s)) for s in range(4)]
    return pl.pallas_call(
        _k4_body,
        grid=(NT,),
        in_specs=in_acc + [
            pl.BlockSpec((ROWS, 64), lambda t: (t, 0)),
            pl.BlockSpec((ROWS, 1), lambda t: (t, 0)),
            pl.BlockSpec((1, 64), lambda t: (0, 0)),
            pl.BlockSpec((64, 32), lambda t: (0, 0)),
        ],
        out_specs=pl.BlockSpec((ROWS, 32), lambda t: (t, 0)),
        out_shape=jax.ShapeDtypeStruct((N, 32), jnp.float32),
    )(acc1, acc1, acc1, acc1, g1flat, dinv, b1.reshape(1, 64), W2)


def _k6_body(a0, a1, g_ref, dinv_ref, b2_ref, z_ref):
    dinv = dinv_ref[...]
    acc = jnp.concatenate([a[...] for a in (a0, a1)], axis=1)
    z_ref[...] = dinv * (acc + g_ref[...]) + b2_ref[...]


def _tc_assemble_z(acc2, g2flat, dinv, b2):
    in_acc = [pl.BlockSpec((ROWS, 16), functools.partial(
        lambda s, t: (s * NST + t, 0), s)) for s in range(2)]
    return pl.pallas_call(
        _k6_body,
        grid=(NT,),
        in_specs=in_acc + [
            pl.BlockSpec((ROWS, 32), lambda t: (t, 0)),
            pl.BlockSpec((ROWS, 1), lambda t: (t, 0)),
            pl.BlockSpec((1, 32), lambda t: (0, 0)),
        ],
        out_specs=pl.BlockSpec((ROWS, 32), lambda t: (t, 0)),
        out_shape=jax.ShapeDtypeStruct((N, 32), jnp.float32),
    )(acc2, acc2, g2flat, dinv, b2.reshape(1, 32))


def _k8_body(s_ref, d_ref, w_ref, bout_ref, o_ref):
    prod = s_ref[...] * d_ref[...] * w_ref[...]
    o_ref[...] = jnp.sum(prod, axis=1, keepdims=True) + bout_ref[...]


def _tc_score(pairs, Wout, bout):
    return pl.pallas_call(
        _k8_body,
        grid=(8,),
        in_specs=[
            pl.BlockSpec((2048, 32), lambda t: (t, 0)),
            pl.BlockSpec((2048, 32), lambda t: (8 + t, 0)),
            pl.BlockSpec((1, 32), lambda t: (0, 0)),
            pl.BlockSpec((1, 1), lambda t: (0, 0)),
        ],
        out_specs=pl.BlockSpec((2048, 1), lambda t: (t, 0)),
        out_shape=jax.ShapeDtypeStruct((16384, 1), jnp.float32),
    )(pairs, pairs, Wout.reshape(1, 32), bout.reshape(1, 1))


@jax.jit
def kernel(x, ei, targets, W1, b1, W2, b2, Wout, bout):
    src, dst = ei[0], ei[1]
    pad = E_PAD - E
    srcp = jnp.concatenate([src, jnp.zeros((pad,), jnp.int32)])
    dstp = jnp.concatenate([dst, jnp.full((pad,), N, jnp.int32)])
    sd1, sd2 = _tc_idx_build(srcp.reshape(NCH_TOT, CHUNK),
                             dstp.reshape(NCH_TOT, CHUNK))

    degpart = _sc_degree(sd1).reshape(2 * NS, 1)

    g1flat, dinv = _tc_scale_l1(x, W1, degpart)
    acc1 = _sc_spmm(g1flat.reshape(4 * N, 16), sd1, 4)
    g2flat = _tc_layer2_tables(acc1, g1flat, dinv, b1, W2)
    acc2 = _sc_spmm(g2flat.reshape(2 * N, 16), sd2, 2)
    z = _tc_assemble_z(acc2, g2flat, dinv, b2)
    pairs = _sc_gather_targets(z, targets.reshape(-1))
    return _tc_score(pairs, Wout, bout)


# speedup vs baseline: 1.1046x; 1.1046x over previous
import functools
import jax
import jax.numpy as jnp
from jax import lax
from jax.experimental import pallas as pl
from jax.experimental.pallas import tpu as pltpu
from jax.experimental.pallas import tpu_sc as plsc

N = 100000
E = 3200000
BLK = 1024
CHUNK = 128
NCHUNK = BLK // CHUNK
E_PAD = 3211264
EP_TILE = E_PAD // 16
NB = EP_TILE // BLK
EP_DEG = E_PAD // 32
NB_DEG = EP_DEG // BLK
NS = 102400
RPT = NS // 16
_ZCHUNKS = tuple((k * BLK, BLK) for k in range(6)) + ((6 * BLK, 256),)

ROWS = 800
NT = N // ROWS
NST = NS // ROWS


def _deg_body(dst_hbm, out_hbm, ones_v, zero_v, dst_v, acc_sh):
    c = lax.axis_index("c")
    tid = lax.axis_index("s")
    row0 = tid * RPT

    def init(k, carry):
        ones_v[pl.ds(k * 16, 16)] = jnp.ones((16,), jnp.float32)
        zero_v[pl.ds(k * 16, 16)] = jnp.zeros((16,), jnp.float32)
        return carry
    lax.fori_loop(0, BLK // 16, init, 0)

    for k0, sz in _ZCHUNKS:
        pltpu.sync_copy(zero_v.at[pl.ds(0, sz)], acc_sh.at[pl.ds(row0 + k0, sz)])
    plsc.subcore_barrier()

    def block(i, carry):
        dbase = (c * 16 + tid) * (EP_DEG // CHUNK) + i * NCHUNK
        pltpu.sync_copy(dst_hbm.at[pl.ds(dbase, NCHUNK)], dst_v)
        for j in range(NCHUNK):
            pltpu.sync_copy(ones_v.at[pl.ds(j * CHUNK, CHUNK)],
                            acc_sh.at[dst_v.at[j]], add=True)
        return carry
    lax.fori_loop(0, NB_DEG, block, 0)
    plsc.subcore_barrier()

    for k0, sz in _ZCHUNKS:
        pltpu.sync_copy(acc_sh.at[pl.ds(row0 + k0, sz)],
                        out_hbm.at[pl.ds(c * NS + row0 + k0, sz)])


def _sc_degree(dst2d):
    kern = pl.kernel(
        _deg_body,
        out_type=jax.ShapeDtypeStruct((2 * NS,), jnp.float32),
        mesh=plsc.VectorSubcoreMesh(core_axis_name="c", subcore_axis_name="s"),
        scratch_types=[
            pltpu.VMEM((BLK,), jnp.float32),
            pltpu.VMEM((BLK,), jnp.float32),
            pltpu.VMEM((NCHUNK, CHUNK), jnp.int32),
            pltpu.VMEM_SHARED((NS,), jnp.float32),
        ],
        compiler_params=pltpu.CompilerParams(use_tc_tiling_on_sc=False),
    )
    return kern(dst2d)


def _spmm_body(spc, g_hbm, src_hbm, dst_hbm, out_hbm,
               src_v, dst_v, rows_v, acc_sh, gsem):
    c = lax.axis_index("c")
    tid = lax.axis_index("s")
    row0 = tid * RPT
    nsl = 2 * spc

    for sl in range(spc):
        slice_id = c * spc + sl

        def zb(k, carry):
            rows_v[k, :] = jnp.zeros((16,), jnp.float32)
            return carry
        lax.fori_loop(0, BLK, zb, 0)
        for k0, sz in _ZCHUNKS:
            pltpu.sync_copy(rows_v.at[pl.ds(0, sz)], acc_sh.at[pl.ds(row0 + k0, sz)])
        plsc.subcore_barrier()

        def block(i, carry):
            base = tid * EP_TILE + i * BLK
            dbase = tid * (EP_TILE // CHUNK) + i * NCHUNK
            pltpu.sync_copy(src_hbm.at[pl.ds(base, BLK)], src_v)
            pltpu.sync_copy(dst_hbm.at[pl.ds(dbase, NCHUNK)], dst_v)
            offv = jnp.full((16,), slice_id, jnp.int32)
            mulv = jnp.full((16,), nsl, jnp.int32)

            def addoff(k, cc):
                src_v[pl.ds(k * 16, 16)] = src_v[pl.ds(k * 16, 16)] * mulv + offv
                return cc
            lax.fori_loop(0, BLK // 16, addoff, 0)

            handles = []
            for j in range(NCHUNK):
                handles.append(pltpu.async_copy(
                    g_hbm.at[src_v.at[pl.ds(j * CHUNK, CHUNK)]],
                    rows_v.at[pl.ds(j * CHUNK, CHUNK)], gsem))
            for h in handles:
                h.wait()
            for j in range(NCHUNK):
                pltpu.sync_copy(rows_v.at[pl.ds(j * CHUNK, CHUNK)],
                                acc_sh.at[dst_v.at[j]], add=True)
            return carry
        lax.fori_loop(0, NB, block, 0)
        plsc.subcore_barrier()

        out_base = slice_id * NS + row0
        for k0, sz in _ZCHUNKS:
            pltpu.sync_copy(acc_sh.at[pl.ds(row0 + k0, sz)],
                            out_hbm.at[pl.ds(out_base + k0, sz)])
        if sl + 1 < spc:
            plsc.subcore_barrier()


def _sc_spmm(g_blocked, srcp, dst2d, n_slices):
    spc = n_slices // 2
    kern = pl.kernel(
        functools.partial(_spmm_body, spc),
        out_type=jax.ShapeDtypeStruct((n_slices * NS, 16), jnp.float32),
        mesh=plsc.VectorSubcoreMesh(core_axis_name="c", subcore_axis_name="s"),
        scratch_types=[
            pltpu.VMEM((BLK,), jnp.int32),
            pltpu.VMEM((NCHUNK, CHUNK), jnp.int32),
            pltpu.VMEM((BLK, 16), jnp.float32),
            pltpu.VMEM_SHARED((NS, 16), jnp.float32),
            pltpu.SemaphoreType.DMA,
        ],
        compiler_params=pltpu.CompilerParams(use_tc_tiling_on_sc=False),
    )
    return kern(g_blocked, srcp, dst2d)


def _tgt_body(z_hbm, t_hbm, out_hbm, idx_v, rows_v, sem):
    c = lax.axis_index("c")
    tid = lax.axis_index("s")
    base = (c * 16 + tid) * BLK
    pltpu.sync_copy(t_hbm.at[pl.ds(base, BLK)], idx_v)
    handles = []
    for j in range(NCHUNK):
        handles.append(pltpu.async_copy(
            z_hbm.at[idx_v.at[pl.ds(j * CHUNK, CHUNK)]],
            rows_v.at[pl.ds(j * CHUNK, CHUNK)], sem))
    for h in handles:
        h.wait()
    pltpu.sync_copy(rows_v, out_hbm.at[pl.ds(base, BLK)])


def _sc_gather_targets(z, tflat):
    kern = pl.kernel(
        _tgt_body,
        out_type=jax.ShapeDtypeStruct((32768, 32), jnp.float32),
        mesh=plsc.VectorSubcoreMesh(core_axis_name="c", subcore_axis_name="s"),
        scratch_types=[
            pltpu.VMEM((BLK,), jnp.int32),
            pltpu.VMEM((BLK, 32), jnp.float32),
            pltpu.SemaphoreType.DMA,
        ],
        compiler_params=pltpu.CompilerParams(use_tc_tiling_on_sc=False),
    )
    return kern(z, tflat)


def _k2_body(x_ref, w_ref, d0_ref, d1_ref, g_ref, dinv_ref):
    dinv = lax.rsqrt(d0_ref[...] + d1_ref[...] + 1.0)
    g_ref[...] = jnp.dot(x_ref[...], w_ref[...],
                         preferred_element_type=jnp.float32) * dinv
    dinv_ref[...] = dinv


def _tc_scale_l1(x, W1, degpart):
    return pl.pallas_call(
        _k2_body,
        grid=(NT,),
        in_specs=[
            pl.BlockSpec((ROWS, 18), lambda t: (t, 0)),
            pl.BlockSpec((18, 64), lambda t: (0, 0)),
            pl.BlockSpec((ROWS, 1), lambda t: (t, 0)),
            pl.BlockSpec((ROWS, 1), lambda t: (NST + t, 0)),
        ],
        out_specs=[
            pl.BlockSpec((ROWS, 64), lambda t: (t, 0)),
            pl.BlockSpec((ROWS, 1), lambda t: (t, 0)),
        ],
        out_shape=[
            jax.ShapeDtypeStruct((N, 64), jnp.float32),
            jax.ShapeDtypeStruct((N, 1), jnp.float32),
        ],
    )(x, W1, degpart, degpart)


def _k4_body(a0, a1, a2, a3, g_ref, dinv_ref, b1_ref, w_ref, out_ref):
    dinv = dinv_ref[...]
    acc = jnp.concatenate([a[...] for a in (a0, a1, a2, a3)], axis=1)
    h = jax.nn.relu(dinv * (acc + g_ref[...]) + b1_ref[...])
    out_ref[...] = jnp.dot(h, w_ref[...], preferred_element_type=jnp.float32) * dinv


def _tc_layer2_tables(acc1, g1flat, dinv, b1, W2):
    in_acc = [pl.BlockSpec((ROWS, 16), functools.partial(
        lambda s, t: (s * NST + t, 0), s)) for s in range(4)]
    return pl.pallas_call(
        _k4_body,
        grid=(NT,),
        in_specs=in_acc + [
            pl.BlockSpec((ROWS, 64), lambda t: (t, 0)),
            pl.BlockSpec((ROWS, 1), lambda t: (t, 0)),
            pl.BlockSpec((1, 64), lambda t: (0, 0)),
            pl.BlockSpec((64, 32), lambda t: (0, 0)),
        ],
        out_specs=pl.BlockSpec((ROWS, 32), lambda t: (t, 0)),
        out_shape=jax.ShapeDtypeStruct((N, 32), jnp.float32),
    )(acc1, acc1, acc1, acc1, g1flat, dinv, b1.reshape(1, 64), W2)


def _k6_body(a0, a1, g_ref, dinv_ref, b2_ref, z_ref):
    dinv = dinv_ref[...]
    acc = jnp.concatenate([a[...] for a in (a0, a1)], axis=1)
    z_ref[...] = dinv * (acc + g_ref[...]) + b2_ref[...]


def _tc_assemble_z(acc2, g2flat, dinv, b2):
    in_acc = [pl.BlockSpec((ROWS, 16), functools.partial(
        lambda s, t: (s * NST + t, 0), s)) for s in range(2)]
    return pl.pallas_call(
        _k6_body,
        grid=(NT,),
        in_specs=in_acc + [
            pl.BlockSpec((ROWS, 32), lambda t: (t, 0)),
            pl.BlockSpec((ROWS, 1), lambda t: (t, 0)),
            pl.BlockSpec((1, 32), lambda t: (0, 0)),
        ],
        out_specs=pl.BlockSpec((ROWS, 32), lambda t: (t, 0)),
        out_shape=jax.ShapeDtypeStruct((N, 32), jnp.float32),
    )(acc2, acc2, g2flat, dinv, b2.reshape(1, 32))


def _k8_body(s_ref, d_ref, w_ref, bout_ref, o_ref):
    prod = s_ref[...] * d_ref[...] * w_ref[...]
    o_ref[...] = jnp.sum(prod, axis=1, keepdims=True) + bout_ref[...]


def _tc_score(pairs, Wout, bout):
    return pl.pallas_call(
        _k8_body,
        grid=(8,),
        in_specs=[
            pl.BlockSpec((2048, 32), lambda t: (t, 0)),
            pl.BlockSpec((2048, 32), lambda t: (8 + t, 0)),
            pl.BlockSpec((1, 32), lambda t: (0, 0)),
            pl.BlockSpec((1, 1), lambda t: (0, 0)),
        ],
        out_specs=pl.BlockSpec((2048, 1), lambda t: (t, 0)),
        out_shape=jax.ShapeDtypeStruct((16384, 1), jnp.float32),
    )(pairs, pairs, Wout.reshape(1, 32), bout.reshape(1, 1))


@jax.jit
def kernel(x, ei, targets, W1, b1, W2, b2, Wout, bout):
    src, dst = ei[0], ei[1]
    pad = E_PAD - E
    srcp = jnp.concatenate([src, jnp.zeros((pad,), jnp.int32)])
    dstp = jnp.concatenate([dst, jnp.full((pad,), N, jnp.int32)])
    dst2d = dstp.reshape(E_PAD // CHUNK, CHUNK)

    degpart = _sc_degree(dst2d).reshape(2 * NS, 1)

    g1flat, dinv = _tc_scale_l1(x, W1, degpart)
    acc1 = _sc_spmm(g1flat.reshape(4 * N, 16), srcp, dst2d, 4)
    g2flat = _tc_layer2_tables(acc1, g1flat, dinv, b1, W2)
    acc2 = _sc_spmm(g2flat.reshape(2 * N, 16), srcp, dst2d, 2)
    z = _tc_assemble_z(acc2, g2flat, dinv, b2)
    pairs = _sc_gather_targets(z, targets.reshape(-1))
    return _tc_score(pairs, Wout, bout)
